# Initial kernel scaffold; baseline (speedup 1.0000x reference)
import jax
import jax.numpy as jnp
from jax.experimental import pallas as pl


def kernel(x_subject, x_roi, ei_s2r, ei_r2r, ei_s2s, c1_s2r_Wl, c1_s2r_bl, c1_s2r_Wr, c1_r2r_Wl, c1_r2r_bl, c1_r2r_Wr, c1_s2s_Wl, c1_s2s_bl, c1_s2s_Wr, c2_s2r_Wl, c2_s2r_bl, c2_s2r_Wr, c2_r2r_Wl, c2_r2r_bl, c2_r2r_Wr, c2_s2s_Wl, c2_s2s_bl, c2_s2s_Wr, lin_W, lin_b):
    raise NotImplementedError("write your pallas kernel here")


# speedup vs baseline: 9.9125x; 9.9125x over previous
import functools

import jax
import jax.numpy as jnp
from jax import lax
from jax.experimental import pallas as pl
from jax.experimental.pallas import tpu as pltpu
from jax.experimental.pallas import tpu_sc as plsc

NS = 10000
E = 640000
D = 128
H = 64
O = 2

NC = 2
NSUB = 16
NW = NC * NSUB
K = 128
CH = -(-E // (NW * K))
E_PAD = NW * K * CH
N_PAD = 10016
CW = 16


def _seg_sum_kernel(with_count):
    mesh = plsc.VectorSubcoreMesh(core_axis_name="c", subcore_axis_name="s")
    out_type = [jax.ShapeDtypeStruct((NC, N_PAD, H), jnp.float32)]
    scratch = [
        pltpu.VMEM((K,), jnp.int32),
        pltpu.VMEM((K,), jnp.int32),
        pltpu.VMEM((K, H), jnp.float32),
        pltpu.VMEM_SHARED((N_PAD, H), jnp.float32),
        pltpu.SemaphoreType.DMA,
    ]
    if with_count:
        out_type.append(jax.ShapeDtypeStruct((NC, N_PAD, CW), jnp.float32))
        scratch += [
            pltpu.VMEM((K, CW), jnp.float32),
            pltpu.VMEM_SHARED((N_PAD, CW), jnp.float32),
        ]

    def body(*refs):
        if with_count:
            (y, srcs, dsts, z64, z16, ones_in,
             acc_out, cnt_out, sidx, didx, rows, acc, sem, ones_v, cnt) = refs
        else:
            (y, srcs, dsts, z64,
             acc_out, sidx, didx, rows, acc, sem) = refs
        cid = lax.axis_index("c")
        sid = lax.axis_index("s")
        wid = cid * NSUB + sid

        @pl.when(sid == 0)
        def _init():
            pltpu.sync_copy(z64, acc)
            if with_count:
                pltpu.sync_copy(z16, cnt)

        if with_count:
            pltpu.sync_copy(ones_in, ones_v)
        plsc.subcore_barrier()

        base0 = wid * (CH * K)

        def chunk(i, carry):
            base = base0 + i * K
            pltpu.sync_copy(srcs.at[pl.ds(base, K)], sidx)
            pltpu.sync_copy(dsts.at[pl.ds(base, K)], didx)
            pltpu.async_copy(y.at[sidx], rows, sem).wait()
            pltpu.sync_copy(rows, acc.at[didx], add=True)
            if with_count:
                pltpu.sync_copy(ones_v, cnt.at[didx], add=True)
            return carry

        lax.fori_loop(0, CH, chunk, 0)
        plsc.subcore_barrier()

        @pl.when(sid == 0)
        def _export():
            pltpu.sync_copy(acc, acc_out.at[cid])
            if with_count:
                pltpu.sync_copy(cnt, cnt_out.at[cid])

    return pl.kernel(
        body,
        out_type=tuple(out_type) if with_count else out_type[0],
        mesh=mesh,
        scratch_types=scratch,
        compiler_params=pltpu.CompilerParams(use_tc_tiling_on_sc=False),
    )


def _pre_body(x_ref, wl_ref, wr_ref, y_ref, z_ref):
    x = x_ref[:]
    y_ref[:] = jnp.dot(x, wl_ref[:], preferred_element_type=jnp.float32)
    z_ref[:] = jnp.dot(x, wr_ref[:], preferred_element_type=jnp.float32)


def _mid_body(sp_ref, cp_ref, z_ref, bl_ref, wl_ref, wr_ref, y2_ref, z2_ref):
    s = sp_ref[0] + sp_ref[1]
    cnt = cp_ref[0, :, 0:1] + cp_ref[1, :, 0:1]
    mean = s / jnp.maximum(cnt, 1.0)
    h = jnp.maximum(mean + bl_ref[:] + z_ref[:], 0.0)
    y2_ref[:] = jnp.dot(h, wl_ref[:], preferred_element_type=jnp.float32)
    z2_ref[:] = jnp.dot(h, wr_ref[:], preferred_element_type=jnp.float32)


def _fin_body(sp_ref, cp_ref, z_ref, bl_ref, wlin_ref, blin_ref, out_ref):
    s = sp_ref[0] + sp_ref[1]
    cnt = cp_ref[0, :, 0:1] + cp_ref[1, :, 0:1]
    mean = s / jnp.maximum(cnt, 1.0)
    h = jnp.maximum(mean + bl_ref[:] + z_ref[:], 0.0)
    out_ref[:] = (
        jnp.dot(h, wlin_ref[:], preferred_element_type=jnp.float32)
        + blin_ref[:]
    )


_f32 = jnp.float32


def kernel(x_subject, x_roi, ei_s2r, ei_r2r, ei_s2s,
           c1_s2r_Wl, c1_s2r_bl, c1_s2r_Wr, c1_r2r_Wl, c1_r2r_bl, c1_r2r_Wr,
           c1_s2s_Wl, c1_s2s_bl, c1_s2s_Wr,
           c2_s2r_Wl, c2_s2r_bl, c2_s2r_Wr, c2_r2r_Wl, c2_r2r_bl, c2_r2r_Wr,
           c2_s2s_Wl, c2_s2s_bl, c2_s2s_Wr,
           lin_W, lin_b):
    xp = jnp.pad(x_subject, ((0, N_PAD - NS), (0, 0)))
    npad = E_PAD - E
    srcs = jnp.concatenate([ei_s2s[0], jnp.full((npad,), NS, jnp.int32)])
    dsts = jnp.concatenate([ei_s2s[1], jnp.full((npad,), N_PAD - 1, jnp.int32)])
    z64 = jnp.zeros((N_PAD, H), _f32)
    z16 = jnp.zeros((N_PAD, CW), _f32)
    ones = jnp.ones((K, CW), _f32)
    bl1 = c1_s2s_bl.reshape(1, H)
    bl2 = c2_s2s_bl.reshape(1, H)
    wlin = jnp.pad(lin_W, ((0, 0), (0, 128 - O)))
    blin = jnp.pad(lin_b, ((0, 128 - O))).reshape(1, 128)

    y1, z1 = pl.pallas_call(
        _pre_body,
        out_shape=[jax.ShapeDtypeStruct((N_PAD, H), _f32)] * 2,
    )(xp, c1_s2s_Wl, c1_s2s_Wr)

    s1p, cntp = _seg_sum_kernel(True)(y1, srcs, dsts, z64, z16, ones)

    y2, z2 = pl.pallas_call(
        _mid_body,
        out_shape=[jax.ShapeDtypeStruct((N_PAD, H), _f32)] * 2,
    )(s1p, cntp, z1, bl1, c2_s2s_Wl, c2_s2s_Wr)

    s2p = _seg_sum_kernel(False)(y2, srcs, dsts, z64)

    outp = pl.pallas_call(
        _fin_body,
        out_shape=jax.ShapeDtypeStruct((N_PAD, 128), _f32),
    )(s2p, cntp, z2, bl2, wlin, blin)

    return outp[:NS, :O]
